# Initial kernel scaffold; baseline (speedup 1.0000x reference)
import jax
import jax.numpy as jnp
from jax.experimental import pallas as pl


def kernel(weight, mask):
    raise NotImplementedError("write your pallas kernel here")


# speedup vs baseline: 2.5806x; 2.5806x over previous
import functools

import jax
import jax.numpy as jnp
from jax.experimental import pallas as pl
from jax.experimental.pallas import tpu as pltpu


def _round_up(x, m):
    return (x + m - 1) // m * m


def _mm_kernel(x_ref, w_ref, o_ref, acc_ref, *, nk, tk, k_tail):
    k = pl.program_id(1)

    @pl.when(k == 0)
    def _():
        acc_ref[...] = jnp.zeros_like(acc_ref)

    x = x_ref[...]
    w = w_ref[...]
    if k_tail != tk:
        limit = jnp.where(k == nk - 1, k_tail, tk)
        xcol = jax.lax.broadcasted_iota(jnp.int32, x.shape, 1)
        wrow = jax.lax.broadcasted_iota(jnp.int32, w.shape, 0)
        x = jnp.where(xcol < limit, x, 0.0)
        w = jnp.where(wrow < limit, w, 0.0)
    acc_ref[...] += jnp.dot(
        x.astype(jnp.bfloat16),
        w.astype(jnp.bfloat16),
        preferred_element_type=jnp.float32,
    )

    @pl.when(k == nk - 1)
    def _():
        o_ref[...] = acc_ref[...].astype(o_ref.dtype)


def kernel(weight, mask):
    B, S, V = mask.shape
    Vw, H = weight.shape
    M = B * S
    x = mask.reshape(M, V)

    tm = None
    for cand in (1024, 512, 256, 128, 64, 32, 16, 8):
        if M % (2 * cand) == 0 and cand <= M // 2:
            tm = cand
            break
    if tm is None:
        Mp = _round_up(M, 8)
        x = jnp.pad(x, ((0, Mp - M), (0, 0)))
        tm = Mp
        M_pad = Mp
    else:
        M_pad = M

    Hp = _round_up(H, 128)
    w = weight if Hp == H else jnp.pad(weight, ((0, 0), (0, Hp - H)))

    tk = 1024
    nk = -(-V // tk)
    k_tail = V - (nk - 1) * tk

    out = pl.pallas_call(
        functools.partial(_mm_kernel, nk=nk, tk=tk, k_tail=k_tail),
        out_shape=jax.ShapeDtypeStruct((M_pad, Hp), weight.dtype),
        grid=(M_pad // tm, nk),
        in_specs=[
            pl.BlockSpec((tm, tk), lambda i, k: (i, k)),
            pl.BlockSpec((tk, Hp), lambda i, k: (k, 0)),
        ],
        out_specs=pl.BlockSpec((tm, Hp), lambda i, k: (i, 0)),
        scratch_shapes=[pltpu.VMEM((tm, Hp), jnp.float32)],
        compiler_params=pltpu.CompilerParams(
            dimension_semantics=("parallel", "arbitrary"),
            vmem_limit_bytes=100 * 1024 * 1024,
        ),
    )(x, w)
    return out[:M, :H].reshape(B, S, H)
